# baseline (speedup 1.0000x reference)
import functools

import jax
import jax.numpy as jnp
from jax import lax
from jax.experimental import pallas as pl
from jax.experimental.pallas import tpu as pltpu
from jax.experimental.pallas import tpu_sc as plsc

_L = 16
_CH = 8192
_NCH = 16
_U = 16


def _chunk_sum_mem(buf, acc, nvec):
  full = nvec // _U

  def body(j, c):
    base = j * (_L * _U)
    for k in range(_U):
      plsc.addupdate(acc.at[pl.ds(k * _L, _L)], buf[pl.ds(base + k * _L, _L)])
    return c

  lax.fori_loop(0, full, body, 0)
  for k in range(nvec - full * _U):
    plsc.addupdate(
        acc.at[pl.ds(k * _L, _L)], buf[pl.ds((full * _U + k) * _L, _L)])


def _tree_sum(accs):
  accs = list(accs)
  while len(accs) > 1:
    accs = [accs[i] + accs[i + 1] for i in range(0, len(accs), 2)]
  return accs[0]


@functools.cache
def _build(n):
  info = plsc.get_sparse_core_info()
  nc = info.num_cores
  nw = nc * info.num_subcores
  per_w = _NCH * _CH
  main_elems = nw * per_w
  extra = (n - main_elems) // _CH
  rem = n - main_elems - extra * _CH
  rem_vecs = (rem + _L - 1) // _L
  assert extra < nw
  mesh = plsc.VectorSubcoreMesh(core_axis_name="c", subcore_axis_name="s")

  @functools.partial(
      pl.kernel,
      mesh=mesh,
      out_type=jax.ShapeDtypeStruct((nw, _L), jnp.float32),
      scratch_types=[
          pltpu.VMEM((_CH,), jnp.float32),
          pltpu.VMEM((_CH,), jnp.float32),
          pltpu.VMEM((_CH,), jnp.float32),
          pltpu.VMEM((_U * _L,), jnp.float32),
          pltpu.VMEM((_L,), jnp.float32),
          pltpu.SemaphoreType.DMA,
          pltpu.SemaphoreType.DMA,
          pltpu.SemaphoreType.DMA,
      ],
  )
  def ksum(vals, out, buf0, buf1, bufx, acc, stage, sem0, sem1, semx):
    wid = lax.axis_index("s") * nc + lax.axis_index("c")
    base = wid * per_w
    zero = jnp.zeros((_L,), jnp.float32)
    bufs = (buf0, buf1)
    sems = (sem0, sem1)

    def start(c, b):
      pltpu.async_copy(
          vals.at[pl.ds(base + c * _CH, _CH)], bufs[b], sems[b])

    def wait(b):
      pltpu.make_async_copy(
          vals.at[pl.ds(0, _CH)], bufs[b], sems[b]).wait()

    if extra:
      @pl.when(wid < extra)
      def _():
        pltpu.async_copy(
            vals.at[pl.ds(main_elems + wid * _CH, _CH)], bufx, semx)
    if rem:
      @pl.when(wid == nw - 1)
      def _():
        bufx[pl.ds(rem_vecs * _L - _L, _L)] = zero
        pltpu.async_copy(
            vals.at[pl.ds(main_elems + extra * _CH, rem)],
            bufx.at[pl.ds(0, rem)], semx)

    for k in range(_U):
      acc[pl.ds(k * _L, _L)] = zero

    start(0, 0)
    start(1, 1)

    def pair_body(p, c):
      c0 = 2 * p
      wait(0)
      _chunk_sum_mem(buf0, acc, _CH // _L)

      @pl.when(c0 + 2 < _NCH)
      def _():
        start(c0 + 2, 0)

      wait(1)
      _chunk_sum_mem(buf1, acc, _CH // _L)

      @pl.when(c0 + 3 < _NCH)
      def _():
        start(c0 + 3, 1)

      return c

    lax.fori_loop(0, _NCH // 2, pair_body, 0)

    if extra:
      @pl.when(wid < extra)
      def _():
        pltpu.make_async_copy(
            vals.at[pl.ds(0, _CH)], bufx, semx).wait()
        _chunk_sum_mem(bufx, acc, _CH // _L)
    if rem:
      @pl.when(wid == nw - 1)
      def _():
        pltpu.make_async_copy(
            vals.at[pl.ds(0, rem)], bufx.at[pl.ds(0, rem)], semx).wait()
        _chunk_sum_mem(bufx, acc, rem_vecs)

    stage[...] = _tree_sum([acc[pl.ds(k * _L, _L)] for k in range(_U)])
    pltpu.sync_copy(stage, out.at[wid])

  return ksum


def kernel(values, indices):
  del indices
  partials = _build(values.shape[0])(values)
  return jnp.sum(partials)


# speedup vs baseline: 1.6268x; 1.6268x over previous
import functools

import jax
import jax.numpy as jnp
from jax import lax
from jax.experimental import pallas as pl
from jax.experimental.pallas import tpu as pltpu
from jax.experimental.pallas import tpu_sc as plsc

_L = 16
_CH = 8192
_NCH = 16
_U = 32
_NACC = 8


def _chunk_sum_reg(buf, accs, nvec):
  accs = list(accs)
  full = nvec // _U

  def body(j, accs):
    accs = list(accs)
    base = j * (_L * _U)
    vs = [buf[pl.ds(base + k * _L, _L)] for k in range(_U)]
    for k in range(_U):
      accs[k % _NACC] = accs[k % _NACC] + vs[k]
    return tuple(accs)

  accs = lax.fori_loop(0, full, body, tuple(accs))
  accs = list(accs)
  for k in range(nvec - full * _U):
    accs[k % _NACC] = accs[k % _NACC] + buf[pl.ds((full * _U + k) * _L, _L)]
  return tuple(accs)


def _tree_sum(accs):
  accs = list(accs)
  while len(accs) > 1:
    accs = [accs[i] + accs[i + 1] for i in range(0, len(accs), 2)]
  return accs[0]


@functools.cache
def _build(n):
  info = plsc.get_sparse_core_info()
  nc = info.num_cores
  nw = nc * info.num_subcores
  per_w = _NCH * _CH
  main_elems = nw * per_w
  extra = (n - main_elems) // _CH
  rem = n - main_elems - extra * _CH
  rem_vecs = (rem + _L - 1) // _L
  assert extra < nw
  mesh = plsc.VectorSubcoreMesh(core_axis_name="c", subcore_axis_name="s")

  @functools.partial(
      pl.kernel,
      mesh=mesh,
      out_type=jax.ShapeDtypeStruct((nw, _L), jnp.float32),
      scratch_types=[
          pltpu.VMEM((_CH,), jnp.float32),
          pltpu.VMEM((_CH,), jnp.float32),
          pltpu.VMEM((_CH,), jnp.float32),
          pltpu.VMEM((_L,), jnp.float32),
          pltpu.SemaphoreType.DMA,
          pltpu.SemaphoreType.DMA,
          pltpu.SemaphoreType.DMA,
      ],
  )
  def ksum(vals, out, buf0, buf1, bufx, stage, sem0, sem1, semx):
    wid = lax.axis_index("s") * nc + lax.axis_index("c")
    base = wid * per_w
    zero = jnp.zeros((_L,), jnp.float32)
    bufs = (buf0, buf1)
    sems = (sem0, sem1)

    def start(c, b):
      pltpu.async_copy(
          vals.at[pl.ds(base + c * _CH, _CH)], bufs[b], sems[b])

    def wait(b):
      pltpu.make_async_copy(
          vals.at[pl.ds(0, _CH)], bufs[b], sems[b]).wait()

    if extra:
      @pl.when(wid < extra)
      def _():
        pltpu.async_copy(
            vals.at[pl.ds(main_elems + wid * _CH, _CH)], bufx, semx)
    if rem:
      @pl.when(wid == nw - 1)
      def _():
        bufx[pl.ds(rem_vecs * _L - _L, _L)] = zero
        pltpu.async_copy(
            vals.at[pl.ds(main_elems + extra * _CH, rem)],
            bufx.at[pl.ds(0, rem)], semx)

    start(0, 0)
    start(1, 1)

    def pair_body(p, accs):
      c0 = 2 * p
      wait(0)
      accs = _chunk_sum_reg(buf0, accs, _CH // _L)

      @pl.when(c0 + 2 < _NCH)
      def _():
        start(c0 + 2, 0)

      wait(1)
      accs = _chunk_sum_reg(buf1, accs, _CH // _L)

      @pl.when(c0 + 3 < _NCH)
      def _():
        start(c0 + 3, 1)

      return accs

    accs = lax.fori_loop(0, _NCH // 2, pair_body, (zero,) * _NACC)
    stage[...] = _tree_sum(accs)

    if extra:
      @pl.when(wid < extra)
      def _():
        pltpu.make_async_copy(
            vals.at[pl.ds(0, _CH)], bufx, semx).wait()
        t = _chunk_sum_reg(bufx, (zero,) * _NACC, _CH // _L)
        stage[...] = stage[...] + _tree_sum(t)
    if rem:
      @pl.when(wid == nw - 1)
      def _():
        pltpu.make_async_copy(
            vals.at[pl.ds(0, rem)], bufx.at[pl.ds(0, rem)], semx).wait()
        t = _chunk_sum_reg(bufx, (zero,) * _NACC, rem_vecs)
        stage[...] = stage[...] + _tree_sum(t)

    pltpu.sync_copy(stage, out.at[wid])

  return ksum


def kernel(values, indices):
  del indices
  partials = _build(values.shape[0])(values)
  return jnp.sum(partials)
